# baseline (speedup 1.0000x reference)
import numpy as np
import jax
import jax.numpy as jnp
from jax.experimental import pallas as pl
from jax.experimental.pallas import tpu as pltpu

_N = 4096
_H = _N // 2


def _dft_mats():
    i = np.arange(_N)
    theta = ((i[:, None].astype(np.int64) * i[None, :]) % _N).astype(np.float64)
    theta *= 2.0 * np.pi / _N
    sgn = np.where(i % 2 == 0, 1.0, -1.0)[:, None]
    c = sgn * np.cos(theta) / _N
    s = sgn * np.sin(theta) / _N
    ce = c[:, :_H].copy()
    se = s[:, :_H].copy()
    ce[:, 0] = sgn[:, 0] / _N
    se[:, 0] = -1.0 / _N
    cte = np.ascontiguousarray(c.T)[:_H, :].copy()
    ste = np.ascontiguousarray(s.T)[:_H, :].copy()
    cte[0, :] = sgn[:, 0] / _N
    ste[0, :] = -1.0 / _N
    bf = jnp.bfloat16
    return ce.astype(bf), se.astype(bf), cte.astype(bf), ste.astype(bf)


_CE, _SE, _CTE, _STE = _dft_mats()

_PW_ROWS = 256
_FC = 128
_BM = 512
_BN = 512
_VMEM = 60 * 1024 * 1024


def _pointwise_body(xr_ref, xi_ref, mk_ref, pk_ref, or_ref, oi_ref):
    xr = xr_ref[...]
    xi = xi_ref[...]
    mag = jnp.sqrt(xr * xr + xi * xi) * mk_ref[...]
    ph = jnp.arctan2(xi, xr) * pk_ref[...]
    or_ref[...] = (mag * jnp.cos(ph)).astype(jnp.bfloat16)
    oi_ref[...] = (mag * jnp.sin(ph)).astype(jnp.bfloat16)


def _lanerev(z):
    idx = (_FC - 1) - jax.lax.broadcasted_iota(jnp.int32, z.shape, 1)
    return jnp.take_along_axis(z, idx, axis=1)


def _colmirror(m1, m2):
    return jnp.concatenate([m2[:, 0:1], _lanerev(m1)[:, : _FC - 1]], axis=1)


def _rowfold(x, xrow, x0, sign):
    row0 = jax.lax.broadcasted_iota(jnp.int32, (_H, _FC), 0) == 0
    folded = x[:_H] + sign * xrow[:_H]
    return jnp.where(row0, x0, folded)


def _fold_body(rk_ref, rm1_ref, rm2_ref, rc_ref,
               rrk_ref, rrm1_ref, rrm2_ref, rrc_ref,
               ik_ref, im1_ref, im2_ref, ic_ref,
               irk_ref, irm1_ref, irm2_ref, irc_ref,
               fa_ref, fb_ref, ga_ref, gb_ref):
    b = pl.program_id(0)
    f32 = jnp.float32
    rk = rk_ref[...].astype(f32)
    ik = ik_ref[...].astype(f32)
    rrk = rrk_ref[...].astype(f32)
    irk = irk_ref[...].astype(f32)
    rv = _colmirror(rm1_ref[...].astype(f32), rm2_ref[...].astype(f32))
    iv = _colmirror(im1_ref[...].astype(f32), im2_ref[...].astype(f32))
    rrv = _colmirror(rrm1_ref[...].astype(f32), rrm2_ref[...].astype(f32))
    irv = _colmirror(irm1_ref[...].astype(f32), irm2_ref[...].astype(f32))

    xre_k = _rowfold(rk, rrk, rk[0:1], 1.0)
    xre_v = _rowfold(rv, rrv, rv[0:1], 1.0)
    xio_k = _rowfold(ik, irk, rk[_H:_H + 1], -1.0)
    xio_v = _rowfold(iv, irv, rv[_H:_H + 1], -1.0)
    xie_k = _rowfold(ik, irk, ik[0:1], 1.0)
    xie_v = _rowfold(iv, irv, iv[0:1], 1.0)
    xro_k = _rowfold(rk, rrk, -ik[_H:_H + 1], -1.0)
    xro_v = _rowfold(rv, rrv, -iv[_H:_H + 1], -1.0)

    fa = xre_k + xre_v
    fb = xio_k + xio_v
    ga = xie_k - xie_v
    gb = xro_k - xro_v

    rc = rc_ref[...].astype(f32)
    ic = ic_ref[...].astype(f32)
    rrc = rrc_ref[...].astype(f32)
    irc = irc_ref[...].astype(f32)
    xre_c = _rowfold(rc, rrc, rc[0:1], 1.0)
    xio_c = _rowfold(ic, irc, rc[_H:_H + 1], -1.0)
    lane0 = jax.lax.broadcasted_iota(jnp.int32, fa.shape, 1) == 0
    mask = jnp.logical_and(lane0, b == 0)
    fa = jnp.where(mask, xre_k[:, 0:1], fa)
    fb = jnp.where(mask, xio_k[:, 0:1], fb)
    ga = jnp.where(mask, xre_c[:, 0:1], ga)
    gb = jnp.where(mask, -xio_c[:, 0:1], gb)

    fa_ref[...] = fa.astype(jnp.bfloat16)
    fb_ref[...] = fb.astype(jnp.bfloat16)
    ga_ref[...] = ga.astype(jnp.bfloat16)
    gb_ref[...] = gb.astype(jnp.bfloat16)


def _stage1_body(ce_ref, se_ref, fa_ref, fb_ref, ga_ref, gb_ref, pe_ref, qo_ref):
    ce = ce_ref[...]
    se = se_ref[...]
    pe_ref[...] = (jnp.dot(ce, fa_ref[...], preferred_element_type=jnp.float32)
                   - jnp.dot(se, fb_ref[...], preferred_element_type=jnp.float32)
                   ).astype(jnp.bfloat16)
    qo_ref[...] = (jnp.dot(ce, ga_ref[...], preferred_element_type=jnp.float32)
                   + jnp.dot(se, gb_ref[...], preferred_element_type=jnp.float32)
                   ).astype(jnp.bfloat16)


def _stage2_body(pe_ref, qo_ref, ct_ref, st_ref, o_ref):
    o_ref[...] = (jnp.dot(pe_ref[...], ct_ref[...], preferred_element_type=jnp.float32)
                  - jnp.dot(qo_ref[...], st_ref[...], preferred_element_type=jnp.float32))


@jax.jit
def kernel(x_real, x_imag, magnitude_kernel, phase_kernel):
    xr = x_real.reshape(_N, _N)
    xi = x_imag.reshape(_N, _N)
    mk = magnitude_kernel.reshape(_N, _N)
    pk = phase_kernel.reshape(_N, _N)

    pw_spec = pl.BlockSpec((_PW_ROWS, _N), lambda i: (i, 0))
    Xr, Xi = pl.pallas_call(
        _pointwise_body,
        grid=(_N // _PW_ROWS,),
        in_specs=[pw_spec] * 4,
        out_specs=[pw_spec] * 2,
        out_shape=[jax.ShapeDtypeStruct((_N, _N), jnp.bfloat16)] * 2,
        compiler_params=pltpu.CompilerParams(
            dimension_semantics=("arbitrary",),
            vmem_limit_bytes=_VMEM,
        ),
    )(xr, xi, mk, pk)

    Xr_row = jnp.roll(jnp.flip(Xr, axis=0), 1, axis=0)
    Xi_row = jnp.roll(jnp.flip(Xi, axis=0), 1, axis=0)

    nfc = _N // _FC
    k_spec = pl.BlockSpec((_N, _FC), lambda b: (0, b))
    m1_spec = pl.BlockSpec((_N, _FC), lambda b: (0, nfc - 1 - b))
    m2_spec = pl.BlockSpec((_N, _FC), lambda b: (0, (nfc - b) % nfc))
    c_spec = pl.BlockSpec((_N, _FC), lambda b: (0, _H // _FC))
    quad = [k_spec, m1_spec, m2_spec, c_spec]
    fold_out = pl.BlockSpec((_H, _FC), lambda b: (0, b))
    F_a, F_b, G_a, G_b = pl.pallas_call(
        _fold_body,
        grid=(_H // _FC,),
        in_specs=quad * 4,
        out_specs=[fold_out] * 4,
        out_shape=[jax.ShapeDtypeStruct((_H, _H), jnp.bfloat16)] * 4,
        compiler_params=pltpu.CompilerParams(
            dimension_semantics=("arbitrary",),
            vmem_limit_bytes=_VMEM,
        ),
    )(Xr, Xr, Xr, Xr, Xr_row, Xr_row, Xr_row, Xr_row,
      Xi, Xi, Xi, Xi, Xi_row, Xi_row, Xi_row, Xi_row)

    lhs1 = pl.BlockSpec((_BM, _H), lambda i, j: (i, 0))
    rhs1 = pl.BlockSpec((_H, _BN), lambda i, j: (0, j))
    out1 = pl.BlockSpec((_BM, _BN), lambda i, j: (i, j))
    pe, qo = pl.pallas_call(
        _stage1_body,
        grid=(_N // _BM, _H // _BN),
        in_specs=[lhs1, lhs1, rhs1, rhs1, rhs1, rhs1],
        out_specs=[out1, out1],
        out_shape=[jax.ShapeDtypeStruct((_N, _H), jnp.bfloat16)] * 2,
        compiler_params=pltpu.CompilerParams(
            dimension_semantics=("arbitrary", "arbitrary"),
            vmem_limit_bytes=_VMEM,
        ),
    )(_CE, _SE, F_a, F_b, G_a, G_b)

    lhs2 = pl.BlockSpec((_BM, _H), lambda i, j: (i, 0))
    rhs2 = pl.BlockSpec((_H, _BN), lambda i, j: (0, j))
    out2 = pl.BlockSpec((_BM, _BN), lambda i, j: (i, j))
    out = pl.pallas_call(
        _stage2_body,
        grid=(_N // _BM, _N // _BN),
        in_specs=[lhs2, lhs2, rhs2, rhs2],
        out_specs=out2,
        out_shape=jax.ShapeDtypeStruct((_N, _N), jnp.float32),
        compiler_params=pltpu.CompilerParams(
            dimension_semantics=("arbitrary", "arbitrary"),
            vmem_limit_bytes=_VMEM,
        ),
    )(pe, qo, _CTE, _STE)

    return out.reshape(1, _N, _N)


# speedup vs baseline: 2.1010x; 2.1010x over previous
import numpy as np
import jax
import jax.numpy as jnp
from jax.experimental import pallas as pl
from jax.experimental.pallas import tpu as pltpu

_N = 4096
_H = _N // 2


def _dft_mats():
    i = np.arange(_N)
    theta = ((i[:, None].astype(np.int64) * i[None, :]) % _N).astype(np.float64)
    theta *= 2.0 * np.pi / _N
    sgn = np.where(i % 2 == 0, 1.0, -1.0)[:, None]
    c = sgn * np.cos(theta) / _N
    s = sgn * np.sin(theta) / _N
    cte = np.ascontiguousarray(c.T)[:_H, :].copy()
    ste = np.ascontiguousarray(s.T)[:_H, :].copy()
    cte[0, :] = sgn[:, 0] / _N
    ste[0, :] = -1.0 / _N
    bf = jnp.bfloat16
    return c.astype(bf), s.astype(bf), cte.astype(bf), ste.astype(bf)


_C, _S, _CTE, _STE = _dft_mats()

_PW_ROWS = 256
_FC = 128
_BM = 512
_BN = 512
_VMEM = 60 * 1024 * 1024


def _pointwise_body(xr_ref, xi_ref, mk_ref, pk_ref, or_ref, oi_ref):
    xr = xr_ref[...]
    xi = xi_ref[...]
    mag = jnp.sqrt(xr * xr + xi * xi) * mk_ref[...]
    ph = jnp.arctan2(xi, xr) * pk_ref[...]
    or_ref[...] = (mag * jnp.cos(ph)).astype(jnp.bfloat16)
    oi_ref[...] = (mag * jnp.sin(ph)).astype(jnp.bfloat16)


def _lanerev(z):
    idx = (_FC - 1) - jax.lax.broadcasted_iota(jnp.int32, z.shape, 1)
    return jnp.take_along_axis(z, idx, axis=1)


def _colmirror(m1, m2):
    return jnp.concatenate([m2[:, 0:1], _lanerev(m1)[:, : _FC - 1]], axis=1)


def _fold_body(rk_ref, rm1_ref, rm2_ref, rc_ref,
               ik_ref, im1_ref, im2_ref, ic_ref,
               fa_ref, fb_ref, ga_ref, gb_ref):
    b = pl.program_id(0)
    f32 = jnp.float32
    rk = rk_ref[...].astype(f32)
    ik = ik_ref[...].astype(f32)
    rv = _colmirror(rm1_ref[...].astype(f32), rm2_ref[...].astype(f32))
    iv = _colmirror(im1_ref[...].astype(f32), im2_ref[...].astype(f32))

    fa = rk + rv
    fb = ik + iv
    ga = ik - iv
    gb = rk - rv

    lane0 = jax.lax.broadcasted_iota(jnp.int32, fa.shape, 1) == 0
    mask = jnp.logical_and(lane0, b == 0)
    fa = jnp.where(mask, rk[:, 0:1], fa)
    fb = jnp.where(mask, ik[:, 0:1], fb)
    ga = jnp.where(mask, rc_ref[:, 0:1].astype(f32), ga)
    gb = jnp.where(mask, -ic_ref[:, 0:1].astype(f32), gb)

    fa_ref[...] = fa.astype(jnp.bfloat16)
    fb_ref[...] = fb.astype(jnp.bfloat16)
    ga_ref[...] = ga.astype(jnp.bfloat16)
    gb_ref[...] = gb.astype(jnp.bfloat16)


def _stage1_body(c_ref, s_ref, fa_ref, fb_ref, ga_ref, gb_ref, pe_ref, qo_ref):
    c = c_ref[...]
    s = s_ref[...]
    pe_ref[...] = (jnp.dot(c, fa_ref[...], preferred_element_type=jnp.float32)
                   - jnp.dot(s, fb_ref[...], preferred_element_type=jnp.float32)
                   ).astype(jnp.bfloat16)
    qo_ref[...] = (jnp.dot(c, ga_ref[...], preferred_element_type=jnp.float32)
                   + jnp.dot(s, gb_ref[...], preferred_element_type=jnp.float32)
                   ).astype(jnp.bfloat16)


def _stage2_body(pe_ref, qo_ref, ct_ref, st_ref, o_ref):
    o_ref[...] = (jnp.dot(pe_ref[...], ct_ref[...], preferred_element_type=jnp.float32)
                  - jnp.dot(qo_ref[...], st_ref[...], preferred_element_type=jnp.float32))


@jax.jit
def kernel(x_real, x_imag, magnitude_kernel, phase_kernel):
    xr = x_real.reshape(_N, _N)
    xi = x_imag.reshape(_N, _N)
    mk = magnitude_kernel.reshape(_N, _N)
    pk = phase_kernel.reshape(_N, _N)

    pw_spec = pl.BlockSpec((_PW_ROWS, _N), lambda i: (i, 0))
    Xr, Xi = pl.pallas_call(
        _pointwise_body,
        grid=(_N // _PW_ROWS,),
        in_specs=[pw_spec] * 4,
        out_specs=[pw_spec] * 2,
        out_shape=[jax.ShapeDtypeStruct((_N, _N), jnp.bfloat16)] * 2,
        compiler_params=pltpu.CompilerParams(
            dimension_semantics=("arbitrary",),
            vmem_limit_bytes=_VMEM,
        ),
    )(xr, xi, mk, pk)

    nfc = _N // _FC
    k_spec = pl.BlockSpec((_N, _FC), lambda b: (0, b))
    m1_spec = pl.BlockSpec((_N, _FC), lambda b: (0, nfc - 1 - b))
    m2_spec = pl.BlockSpec((_N, _FC), lambda b: (0, (nfc - b) % nfc))
    c_spec = pl.BlockSpec((_N, _FC), lambda b: (0, _H // _FC))
    quad = [k_spec, m1_spec, m2_spec, c_spec]
    fold_out = pl.BlockSpec((_N, _FC), lambda b: (0, b))
    F_a, F_b, G_a, G_b = pl.pallas_call(
        _fold_body,
        grid=(_H // _FC,),
        in_specs=quad * 2,
        out_specs=[fold_out] * 4,
        out_shape=[jax.ShapeDtypeStruct((_N, _H), jnp.bfloat16)] * 4,
        compiler_params=pltpu.CompilerParams(
            dimension_semantics=("arbitrary",),
            vmem_limit_bytes=_VMEM,
        ),
    )(Xr, Xr, Xr, Xr, Xi, Xi, Xi, Xi)

    lhs1 = pl.BlockSpec((_BM, _N), lambda i, j: (i, 0))
    rhs1 = pl.BlockSpec((_N, _BN), lambda i, j: (0, j))
    out1 = pl.BlockSpec((_BM, _BN), lambda i, j: (i, j))
    pe, qo = pl.pallas_call(
        _stage1_body,
        grid=(_N // _BM, _H // _BN),
        in_specs=[lhs1, lhs1, rhs1, rhs1, rhs1, rhs1],
        out_specs=[out1, out1],
        out_shape=[jax.ShapeDtypeStruct((_N, _H), jnp.bfloat16)] * 2,
        compiler_params=pltpu.CompilerParams(
            dimension_semantics=("arbitrary", "arbitrary"),
            vmem_limit_bytes=_VMEM,
        ),
    )(_C, _S, F_a, F_b, G_a, G_b)

    lhs2 = pl.BlockSpec((_BM, _H), lambda i, j: (i, 0))
    rhs2 = pl.BlockSpec((_H, _BN), lambda i, j: (0, j))
    out2 = pl.BlockSpec((_BM, _BN), lambda i, j: (i, j))
    out = pl.pallas_call(
        _stage2_body,
        grid=(_N // _BM, _N // _BN),
        in_specs=[lhs2, lhs2, rhs2, rhs2],
        out_specs=out2,
        out_shape=jax.ShapeDtypeStruct((_N, _N), jnp.float32),
        compiler_params=pltpu.CompilerParams(
            dimension_semantics=("arbitrary", "arbitrary"),
            vmem_limit_bytes=_VMEM,
        ),
    )(pe, qo, _CTE, _STE)

    return out.reshape(1, _N, _N)


# speedup vs baseline: 2.6035x; 1.2392x over previous
import numpy as np
import jax
import jax.numpy as jnp
from jax.experimental import pallas as pl
from jax.experimental.pallas import tpu as pltpu

_N = 4096
_H = _N // 2


def _dft_mats():
    i = np.arange(_N)
    theta = ((i[:, None].astype(np.int64) * i[None, :]) % _N).astype(np.float64)
    theta *= 2.0 * np.pi / _N
    sgn = np.where(i % 2 == 0, 1.0, -1.0)[:, None]
    c = sgn * np.cos(theta) / _N
    s = sgn * np.sin(theta) / _N
    cte = np.ascontiguousarray(c.T)[:_H, :].copy()
    ste = np.ascontiguousarray(s.T)[:_H, :].copy()
    cte[0, :] = sgn[:, 0] / _N
    ste[0, :] = -1.0 / _N
    bf = jnp.bfloat16
    return c.astype(bf), s.astype(bf), cte.astype(bf), ste.astype(bf)


_C, _S, _CTE, _STE = _dft_mats()

_PW_ROWS = 256
_FC = 128
_BM = 512
_BN = 512
_VMEM = 60 * 1024 * 1024


def _fit_even_poly(f, lo, hi, deg):
    xs = np.cos(np.pi * (np.arange(4096) + 0.5) / 4096)
    zs = lo + (hi - lo) * (xs + 1.0) / 2.0
    out = np.polyfit(zs, f(zs), deg)
    return [np.float32(v) for v in out]


_ATAN_Z = _fit_even_poly(
    lambda z: np.arctan(np.sqrt(z)) / np.sqrt(z), 1e-8, 1.0, 10)
_SIN_Z = _fit_even_poly(
    lambda v: np.sin(2 * np.pi * np.sqrt(v)) / np.sqrt(v), 1e-10, 0.25, 7)
_COS_Z = _fit_even_poly(
    lambda v: np.cos(2 * np.pi * np.sqrt(v)), 1e-10, 0.25, 7)


def _horner(coeffs, z):
    p = jnp.float32(coeffs[0])
    for cc in coeffs[1:]:
        p = p * z + jnp.float32(cc)
    return p


def _fast_atan2(y, x):
    ax = jnp.abs(x)
    ay = jnp.abs(y)
    hi = jnp.maximum(ax, ay)
    lo = jnp.minimum(ax, ay)
    safe = jnp.where(hi > 0.0, hi, 1.0)
    t = lo * jnp.where(hi > 0.0, 1.0 / safe, 0.0)
    z = t * t
    r = t * _horner(_ATAN_Z, z)
    r = jnp.where(ay > ax, jnp.float32(np.pi / 2) - r, r)
    r = jnp.where(x < 0.0, jnp.float32(np.pi) - r, r)
    return jnp.where(y < 0.0, -r, r)


def _pointwise_body(xr_ref, xi_ref, mk_ref, pk_ref, or_ref, oi_ref):
    xr = xr_ref[...]
    xi = xi_ref[...]
    mag = jnp.sqrt(xr * xr + xi * xi) * mk_ref[...]
    ph = _fast_atan2(xi, xr) * pk_ref[...]
    u = ph * jnp.float32(1.0 / (2.0 * np.pi))
    w = u - jnp.round(u)
    v = w * w
    sinv = w * _horner(_SIN_Z, v)
    cosv = _horner(_COS_Z, v)
    or_ref[...] = (mag * cosv).astype(jnp.bfloat16)
    oi_ref[...] = (mag * sinv).astype(jnp.bfloat16)


def _lanerev(z):
    idx = (_FC - 1) - jax.lax.broadcasted_iota(jnp.int32, z.shape, 1)
    return jnp.take_along_axis(z, idx, axis=1)


def _colmirror(m1, m2):
    return jnp.concatenate([m2[:, 0:1], _lanerev(m1)[:, : _FC - 1]], axis=1)


def _fold_body(rk_ref, rm1_ref, rm2_ref, rc_ref,
               ik_ref, im1_ref, im2_ref, ic_ref,
               fa_ref, fb_ref, ga_ref, gb_ref):
    b = pl.program_id(0)
    f32 = jnp.float32
    rk = rk_ref[...].astype(f32)
    ik = ik_ref[...].astype(f32)
    rv = _colmirror(rm1_ref[...].astype(f32), rm2_ref[...].astype(f32))
    iv = _colmirror(im1_ref[...].astype(f32), im2_ref[...].astype(f32))

    fa = rk + rv
    fb = ik + iv
    ga = ik - iv
    gb = rk - rv

    lane0 = jax.lax.broadcasted_iota(jnp.int32, fa.shape, 1) == 0
    mask = jnp.logical_and(lane0, b == 0)
    fa = jnp.where(mask, rk[:, 0:1], fa)
    fb = jnp.where(mask, ik[:, 0:1], fb)
    ga = jnp.where(mask, rc_ref[:, 0:1].astype(f32), ga)
    gb = jnp.where(mask, -ic_ref[:, 0:1].astype(f32), gb)

    fa_ref[...] = fa.astype(jnp.bfloat16)
    fb_ref[...] = fb.astype(jnp.bfloat16)
    ga_ref[...] = ga.astype(jnp.bfloat16)
    gb_ref[...] = gb.astype(jnp.bfloat16)


def _stage1_body(c_ref, s_ref, fa_ref, fb_ref, ga_ref, gb_ref, pe_ref, qo_ref):
    c = c_ref[...]
    s = s_ref[...]
    pe_ref[...] = (jnp.dot(c, fa_ref[...], preferred_element_type=jnp.float32)
                   - jnp.dot(s, fb_ref[...], preferred_element_type=jnp.float32)
                   ).astype(jnp.bfloat16)
    qo_ref[...] = (jnp.dot(c, ga_ref[...], preferred_element_type=jnp.float32)
                   + jnp.dot(s, gb_ref[...], preferred_element_type=jnp.float32)
                   ).astype(jnp.bfloat16)


def _stage2_body(pe_ref, qo_ref, ct_ref, st_ref, o_ref):
    o_ref[...] = (jnp.dot(pe_ref[...], ct_ref[...], preferred_element_type=jnp.float32)
                  - jnp.dot(qo_ref[...], st_ref[...], preferred_element_type=jnp.float32))


@jax.jit
def kernel(x_real, x_imag, magnitude_kernel, phase_kernel):
    xr = x_real.reshape(_N, _N)
    xi = x_imag.reshape(_N, _N)
    mk = magnitude_kernel.reshape(_N, _N)
    pk = phase_kernel.reshape(_N, _N)

    pw_spec = pl.BlockSpec((_PW_ROWS, _N), lambda i: (i, 0))
    Xr, Xi = pl.pallas_call(
        _pointwise_body,
        grid=(_N // _PW_ROWS,),
        in_specs=[pw_spec] * 4,
        out_specs=[pw_spec] * 2,
        out_shape=[jax.ShapeDtypeStruct((_N, _N), jnp.bfloat16)] * 2,
        compiler_params=pltpu.CompilerParams(
            dimension_semantics=("arbitrary",),
            vmem_limit_bytes=_VMEM,
        ),
    )(xr, xi, mk, pk)

    nfc = _N // _FC
    k_spec = pl.BlockSpec((_N, _FC), lambda b: (0, b))
    m1_spec = pl.BlockSpec((_N, _FC), lambda b: (0, nfc - 1 - b))
    m2_spec = pl.BlockSpec((_N, _FC), lambda b: (0, (nfc - b) % nfc))
    c_spec = pl.BlockSpec((_N, _FC), lambda b: (0, _H // _FC))
    quad = [k_spec, m1_spec, m2_spec, c_spec]
    fold_out = pl.BlockSpec((_N, _FC), lambda b: (0, b))
    F_a, F_b, G_a, G_b = pl.pallas_call(
        _fold_body,
        grid=(_H // _FC,),
        in_specs=quad * 2,
        out_specs=[fold_out] * 4,
        out_shape=[jax.ShapeDtypeStruct((_N, _H), jnp.bfloat16)] * 4,
        compiler_params=pltpu.CompilerParams(
            dimension_semantics=("arbitrary",),
            vmem_limit_bytes=_VMEM,
        ),
    )(Xr, Xr, Xr, Xr, Xi, Xi, Xi, Xi)

    lhs1 = pl.BlockSpec((_BM, _N), lambda i, j: (i, 0))
    rhs1 = pl.BlockSpec((_N, _BN), lambda i, j: (0, j))
    out1 = pl.BlockSpec((_BM, _BN), lambda i, j: (i, j))
    pe, qo = pl.pallas_call(
        _stage1_body,
        grid=(_N // _BM, _H // _BN),
        in_specs=[lhs1, lhs1, rhs1, rhs1, rhs1, rhs1],
        out_specs=[out1, out1],
        out_shape=[jax.ShapeDtypeStruct((_N, _H), jnp.bfloat16)] * 2,
        compiler_params=pltpu.CompilerParams(
            dimension_semantics=("arbitrary", "arbitrary"),
            vmem_limit_bytes=_VMEM,
        ),
    )(_C, _S, F_a, F_b, G_a, G_b)

    lhs2 = pl.BlockSpec((_BM, _H), lambda i, j: (i, 0))
    rhs2 = pl.BlockSpec((_H, _BN), lambda i, j: (0, j))
    out2 = pl.BlockSpec((_BM, _BN), lambda i, j: (i, j))
    out = pl.pallas_call(
        _stage2_body,
        grid=(_N // _BM, _N // _BN),
        in_specs=[lhs2, lhs2, rhs2, rhs2],
        out_specs=out2,
        out_shape=jax.ShapeDtypeStruct((_N, _N), jnp.float32),
        compiler_params=pltpu.CompilerParams(
            dimension_semantics=("arbitrary", "arbitrary"),
            vmem_limit_bytes=_VMEM,
        ),
    )(pe, qo, _CTE, _STE)

    return out.reshape(1, _N, _N)


# speedup vs baseline: 2.6721x; 1.0264x over previous
import numpy as np
import jax
import jax.numpy as jnp
from jax.experimental import pallas as pl
from jax.experimental.pallas import tpu as pltpu

_N = 4096
_H = _N // 2


def _dft_mats():
    i = np.arange(_N)
    theta = ((i[:, None].astype(np.int64) * i[None, :]) % _N).astype(np.float64)
    theta *= 2.0 * np.pi / _N
    sgn = np.where(i % 2 == 0, 1.0, -1.0)[:, None]
    c = sgn * np.cos(theta) / _N
    s = sgn * np.sin(theta) / _N
    cte = np.ascontiguousarray(c.T)[:_H, :].copy()
    ste = np.ascontiguousarray(s.T)[:_H, :].copy()
    cte[0, :] = sgn[:, 0] / _N
    ste[0, :] = -1.0 / _N
    bf = jnp.bfloat16
    return c.astype(bf), s.astype(bf), cte.astype(bf), ste.astype(bf)


_C, _S, _CTE, _STE = _dft_mats()

_PW_ROWS = 256
_FC = 128
_BM = 512
_BN = 512
_VMEM = 60 * 1024 * 1024


def _fit_even_poly(f, lo, hi, deg):
    xs = np.cos(np.pi * (np.arange(4096) + 0.5) / 4096)
    zs = lo + (hi - lo) * (xs + 1.0) / 2.0
    out = np.polyfit(zs, f(zs), deg)
    return [np.float32(v) for v in out]


_ATAN_Z = _fit_even_poly(
    lambda z: np.arctan(np.sqrt(z)) / np.sqrt(z), 1e-8, 1.0, 10)
_SIN_Z = _fit_even_poly(
    lambda v: np.sin(2 * np.pi * np.sqrt(v)) / np.sqrt(v), 1e-10, 0.25, 7)
_COS_Z = _fit_even_poly(
    lambda v: np.cos(2 * np.pi * np.sqrt(v)), 1e-10, 0.25, 7)


def _horner(coeffs, z):
    p = jnp.float32(coeffs[0])
    for cc in coeffs[1:]:
        p = p * z + jnp.float32(cc)
    return p


def _fast_atan2(y, x):
    ax = jnp.abs(x)
    ay = jnp.abs(y)
    hi = jnp.maximum(ax, ay)
    lo = jnp.minimum(ax, ay)
    safe = jnp.where(hi > 0.0, hi, 1.0)
    t = lo * jnp.where(hi > 0.0, 1.0 / safe, 0.0)
    z = t * t
    r = t * _horner(_ATAN_Z, z)
    r = jnp.where(ay > ax, jnp.float32(np.pi / 2) - r, r)
    r = jnp.where(x < 0.0, jnp.float32(np.pi) - r, r)
    return jnp.where(y < 0.0, -r, r)


def _pointwise_body(xr_ref, xi_ref, mk_ref, pk_ref, or_ref, oi_ref):
    xr = xr_ref[...]
    xi = xi_ref[...]
    mag = jnp.sqrt(xr * xr + xi * xi) * mk_ref[...]
    ph = _fast_atan2(xi, xr) * pk_ref[...]
    u = ph * jnp.float32(1.0 / (2.0 * np.pi))
    w = u - jnp.round(u)
    v = w * w
    sinv = w * _horner(_SIN_Z, v)
    cosv = _horner(_COS_Z, v)
    or_ref[...] = (mag * cosv).astype(jnp.bfloat16)
    oi_ref[...] = (mag * sinv).astype(jnp.bfloat16)


def _lanerev(z):
    idx = (_FC - 1) - jax.lax.broadcasted_iota(jnp.int32, z.shape, 1)
    return jnp.take_along_axis(z, idx, axis=1)


def _colmirror(m1, m2):
    return jnp.concatenate([m2[:, 0:1], _lanerev(m1)[:, : _FC - 1]], axis=1)


def _fold_body(rk_ref, rm1_ref, rm2_ref, rc_ref,
               ik_ref, im1_ref, im2_ref, ic_ref,
               fa_ref, fb_ref, ga_ref, gb_ref):
    b = pl.program_id(0)
    f32 = jnp.float32
    rk = rk_ref[...].astype(f32)
    ik = ik_ref[...].astype(f32)
    rv = _colmirror(rm1_ref[...].astype(f32), rm2_ref[...].astype(f32))
    iv = _colmirror(im1_ref[...].astype(f32), im2_ref[...].astype(f32))

    fa = rk + rv
    fb = ik + iv
    ga = ik - iv
    gb = rk - rv

    lane0 = jax.lax.broadcasted_iota(jnp.int32, fa.shape, 1) == 0
    mask = jnp.logical_and(lane0, b == 0)
    fa = jnp.where(mask, rk[:, 0:1], fa)
    fb = jnp.where(mask, ik[:, 0:1], fb)
    ga = jnp.where(mask, rc_ref[:, 0:1].astype(f32), ga)
    gb = jnp.where(mask, -ic_ref[:, 0:1].astype(f32), gb)

    fa_ref[...] = fa.astype(jnp.bfloat16)
    fb_ref[...] = fb.astype(jnp.bfloat16)
    ga_ref[...] = ga.astype(jnp.bfloat16)
    gb_ref[...] = gb.astype(jnp.bfloat16)


def _stage1_body(c_ref, s_ref, fa_ref, fb_ref, ga_ref, gb_ref, pe_ref, qo_ref):
    c = c_ref[...]
    s = s_ref[...]
    pe_ref[...] = (jnp.dot(c, fa_ref[...], preferred_element_type=jnp.float32)
                   - jnp.dot(s, fb_ref[...], preferred_element_type=jnp.float32)
                   ).astype(jnp.bfloat16)
    qo_ref[...] = (jnp.dot(c, ga_ref[...], preferred_element_type=jnp.float32)
                   + jnp.dot(s, gb_ref[...], preferred_element_type=jnp.float32)
                   ).astype(jnp.bfloat16)


def _stage2_body(pe_ref, qo_ref, ct_ref, st_ref, o_ref):
    o_ref[...] = (jnp.dot(pe_ref[...], ct_ref[...], preferred_element_type=jnp.float32)
                  - jnp.dot(qo_ref[...], st_ref[...], preferred_element_type=jnp.float32))


@jax.jit
def kernel(x_real, x_imag, magnitude_kernel, phase_kernel):
    xr = x_real.reshape(_N, _N)
    xi = x_imag.reshape(_N, _N)
    mk = magnitude_kernel.reshape(_N, _N)
    pk = phase_kernel.reshape(_N, _N)

    pw_spec = pl.BlockSpec((_PW_ROWS, _N), lambda i: (i, 0))
    Xr, Xi = pl.pallas_call(
        _pointwise_body,
        grid=(_N // _PW_ROWS,),
        in_specs=[pw_spec] * 4,
        out_specs=[pw_spec] * 2,
        out_shape=[jax.ShapeDtypeStruct((_N, _N), jnp.bfloat16)] * 2,
        compiler_params=pltpu.CompilerParams(
            dimension_semantics=("arbitrary",),
            vmem_limit_bytes=_VMEM,
        ),
    )(xr, xi, mk, pk)

    nfc = _N // _FC
    k_spec = pl.BlockSpec((_N, _FC), lambda b: (0, b))
    m1_spec = pl.BlockSpec((_N, _FC), lambda b: (0, nfc - 1 - b))
    m2_spec = pl.BlockSpec((_N, _FC), lambda b: (0, (nfc - b) % nfc))
    c_spec = pl.BlockSpec((_N, _FC), lambda b: (0, _H // _FC))
    quad = [k_spec, m1_spec, m2_spec, c_spec]
    fold_out = pl.BlockSpec((_N, _FC), lambda b: (0, b))
    F_a, F_b, G_a, G_b = pl.pallas_call(
        _fold_body,
        grid=(_H // _FC,),
        in_specs=quad * 2,
        out_specs=[fold_out] * 4,
        out_shape=[jax.ShapeDtypeStruct((_N, _H), jnp.bfloat16)] * 4,
        compiler_params=pltpu.CompilerParams(
            dimension_semantics=("arbitrary",),
            vmem_limit_bytes=_VMEM,
        ),
    )(Xr, Xr, Xr, Xr, Xi, Xi, Xi, Xi)

    lhs1 = pl.BlockSpec((_BM, _N), lambda j, i: (i, 0))
    rhs1 = pl.BlockSpec((_N, _BN), lambda j, i: (0, j))
    out1 = pl.BlockSpec((_BM, _BN), lambda j, i: (i, j))
    pe, qo = pl.pallas_call(
        _stage1_body,
        grid=(_H // _BN, _N // _BM),
        in_specs=[lhs1, lhs1, rhs1, rhs1, rhs1, rhs1],
        out_specs=[out1, out1],
        out_shape=[jax.ShapeDtypeStruct((_N, _H), jnp.bfloat16)] * 2,
        compiler_params=pltpu.CompilerParams(
            dimension_semantics=("arbitrary", "arbitrary"),
            vmem_limit_bytes=_VMEM,
        ),
    )(_C, _S, F_a, F_b, G_a, G_b)

    bm2 = 1024
    lhs2 = pl.BlockSpec((bm2, _H), lambda i, j: (i, 0))
    rhs2 = pl.BlockSpec((_H, _BN), lambda i, j: (0, j))
    out2 = pl.BlockSpec((bm2, _BN), lambda i, j: (i, j))
    out = pl.pallas_call(
        _stage2_body,
        grid=(_N // bm2, _N // _BN),
        in_specs=[lhs2, lhs2, rhs2, rhs2],
        out_specs=out2,
        out_shape=jax.ShapeDtypeStruct((_N, _N), jnp.float32),
        compiler_params=pltpu.CompilerParams(
            dimension_semantics=("arbitrary", "arbitrary"),
            vmem_limit_bytes=_VMEM,
        ),
    )(pe, qo, _CTE, _STE)

    return out.reshape(1, _N, _N)


# speedup vs baseline: 3.0268x; 1.1327x over previous
import numpy as np
import jax
import jax.numpy as jnp
from jax.experimental import pallas as pl
from jax.experimental.pallas import tpu as pltpu

_N = 4096
_H = _N // 2


def _dft_mats():
    i = np.arange(_N)
    theta = ((i[:, None].astype(np.int64) * i[None, :]) % _N).astype(np.float64)
    theta *= 2.0 * np.pi / _N
    sgn = np.where(i % 2 == 0, 1.0, -1.0)[:, None]
    c = sgn * np.cos(theta) / _N
    s = sgn * np.sin(theta) / _N
    cte = np.ascontiguousarray(c.T)[:_H, :].copy()
    ste = np.ascontiguousarray(s.T)[:_H, :].copy()
    cte[0, :] = sgn[:, 0] / _N
    ste[0, :] = -1.0 / _N
    ce = c[:, :_H].copy()
    se = s[:, :_H].copy()
    ce[:, 0] = sgn[:, 0] / _N
    se[:, 0] = -1.0 / _N
    bf = jnp.bfloat16
    return ce.astype(bf), se.astype(bf), cte.astype(bf), ste.astype(bf)


_CE, _SE, _CTE, _STE = _dft_mats()

_PW_ROWS = 256
_FC = 128
_BM = 512
_BN = 512
_VMEM = 60 * 1024 * 1024


def _fit_even_poly(f, lo, hi, deg):
    xs = np.cos(np.pi * (np.arange(4096) + 0.5) / 4096)
    zs = lo + (hi - lo) * (xs + 1.0) / 2.0
    out = np.polyfit(zs, f(zs), deg)
    return [np.float32(v) for v in out]


_ATAN_Z = _fit_even_poly(
    lambda z: np.arctan(np.sqrt(z)) / np.sqrt(z), 1e-8, 1.0, 10)
_SIN_Z = _fit_even_poly(
    lambda v: np.sin(2 * np.pi * np.sqrt(v)) / np.sqrt(v), 1e-10, 0.25, 7)
_COS_Z = _fit_even_poly(
    lambda v: np.cos(2 * np.pi * np.sqrt(v)), 1e-10, 0.25, 7)


def _horner(coeffs, z):
    p = jnp.float32(coeffs[0])
    for cc in coeffs[1:]:
        p = p * z + jnp.float32(cc)
    return p


def _fast_atan2(y, x):
    ax = jnp.abs(x)
    ay = jnp.abs(y)
    hi = jnp.maximum(ax, ay)
    lo = jnp.minimum(ax, ay)
    safe = jnp.where(hi > 0.0, hi, 1.0)
    t = lo * jnp.where(hi > 0.0, 1.0 / safe, 0.0)
    z = t * t
    r = t * _horner(_ATAN_Z, z)
    r = jnp.where(ay > ax, jnp.float32(np.pi / 2) - r, r)
    r = jnp.where(x < 0.0, jnp.float32(np.pi) - r, r)
    return jnp.where(y < 0.0, -r, r)


def _pointwise_body(xr_ref, xi_ref, mk_ref, pk_ref, or_ref, oi_ref):
    xr = xr_ref[...]
    xi = xi_ref[...]
    mag = jnp.sqrt(xr * xr + xi * xi) * mk_ref[...]
    ph = _fast_atan2(xi, xr) * pk_ref[...]
    u = ph * jnp.float32(1.0 / (2.0 * np.pi))
    w = u - jnp.round(u)
    v = w * w
    sinv = w * _horner(_SIN_Z, v)
    cosv = _horner(_COS_Z, v)
    or_ref[...] = (mag * cosv).astype(jnp.bfloat16)
    oi_ref[...] = (mag * sinv).astype(jnp.bfloat16)


def _lanerev(z):
    idx = (_FC - 1) - jax.lax.broadcasted_iota(jnp.int32, z.shape, 1)
    return jnp.take_along_axis(z, idx, axis=1)


def _colmirror(m1, m2):
    return jnp.concatenate([m2[:, 0:1], _lanerev(m1)[:, : _FC - 1]], axis=1)


def _fold_body(rk_ref, rm1_ref, rm2_ref, rc_ref,
               ik_ref, im1_ref, im2_ref, ic_ref,
               fa_ref, fb_ref, ga_ref, gb_ref):
    b = pl.program_id(0)
    f32 = jnp.float32
    rk = rk_ref[...].astype(f32)
    ik = ik_ref[...].astype(f32)
    rv = _colmirror(rm1_ref[...].astype(f32), rm2_ref[...].astype(f32))
    iv = _colmirror(im1_ref[...].astype(f32), im2_ref[...].astype(f32))

    fa = rk + rv
    fb = ik + iv
    ga = ik - iv
    gb = rk - rv

    lane0 = jax.lax.broadcasted_iota(jnp.int32, fa.shape, 1) == 0
    mask = jnp.logical_and(lane0, b == 0)
    fa = jnp.where(mask, rk[:, 0:1], fa)
    fb = jnp.where(mask, ik[:, 0:1], fb)
    ga = jnp.where(mask, rc_ref[:, 0:1].astype(f32), ga)
    gb = jnp.where(mask, -ic_ref[:, 0:1].astype(f32), gb)

    def rowrev(z):
        tiles = []
        for t in range(15, -1, -1):
            tt = z[128 * t:128 * (t + 1)].T
            tiles.append(_lanerev(tt).T)
        return jnp.concatenate(tiles, axis=0)

    mfa = rowrev(fa[_H:])
    mfb = rowrev(fb[_H:])
    mga = rowrev(ga[_H:])
    mgb = rowrev(gb[_H:])

    fa_ref[...] = jnp.concatenate(
        [fa[0:1], fa[1:_H] + mfa[: _H - 1]], axis=0).astype(jnp.bfloat16)
    fb_ref[...] = jnp.concatenate(
        [fa[_H:_H + 1], fb[1:_H] - mfb[: _H - 1]], axis=0).astype(jnp.bfloat16)
    ga_ref[...] = jnp.concatenate(
        [ga[0:1], ga[1:_H] + mga[: _H - 1]], axis=0).astype(jnp.bfloat16)
    gb_ref[...] = jnp.concatenate(
        [-ga[_H:_H + 1], gb[1:_H] - mgb[: _H - 1]], axis=0).astype(jnp.bfloat16)


def _stage1_body(c_ref, s_ref, fa_ref, fb_ref, ga_ref, gb_ref, pe_ref, qo_ref):
    c = c_ref[...]
    s = s_ref[...]
    pe_ref[...] = (jnp.dot(c, fa_ref[...], preferred_element_type=jnp.float32)
                   - jnp.dot(s, fb_ref[...], preferred_element_type=jnp.float32)
                   ).astype(jnp.bfloat16)
    qo_ref[...] = (jnp.dot(c, ga_ref[...], preferred_element_type=jnp.float32)
                   + jnp.dot(s, gb_ref[...], preferred_element_type=jnp.float32)
                   ).astype(jnp.bfloat16)


def _stage2_body(pe_ref, qo_ref, ct_ref, st_ref, o_ref):
    o_ref[...] = (jnp.dot(pe_ref[...], ct_ref[...], preferred_element_type=jnp.float32)
                  - jnp.dot(qo_ref[...], st_ref[...], preferred_element_type=jnp.float32))


@jax.jit
def kernel(x_real, x_imag, magnitude_kernel, phase_kernel):
    xr = x_real.reshape(_N, _N)
    xi = x_imag.reshape(_N, _N)
    mk = magnitude_kernel.reshape(_N, _N)
    pk = phase_kernel.reshape(_N, _N)

    pw_spec = pl.BlockSpec((_PW_ROWS, _N), lambda i: (i, 0))
    Xr, Xi = pl.pallas_call(
        _pointwise_body,
        grid=(_N // _PW_ROWS,),
        in_specs=[pw_spec] * 4,
        out_specs=[pw_spec] * 2,
        out_shape=[jax.ShapeDtypeStruct((_N, _N), jnp.bfloat16)] * 2,
        compiler_params=pltpu.CompilerParams(
            dimension_semantics=("arbitrary",),
            vmem_limit_bytes=_VMEM,
        ),
    )(xr, xi, mk, pk)

    nfc = _N // _FC
    k_spec = pl.BlockSpec((_N, _FC), lambda b: (0, b))
    m1_spec = pl.BlockSpec((_N, _FC), lambda b: (0, nfc - 1 - b))
    m2_spec = pl.BlockSpec((_N, _FC), lambda b: (0, (nfc - b) % nfc))
    c_spec = pl.BlockSpec((_N, _FC), lambda b: (0, _H // _FC))
    quad = [k_spec, m1_spec, m2_spec, c_spec]
    fold_out = pl.BlockSpec((_H, _FC), lambda b: (0, b))
    F_a, F_b, G_a, G_b = pl.pallas_call(
        _fold_body,
        grid=(_H // _FC,),
        in_specs=quad * 2,
        out_specs=[fold_out] * 4,
        out_shape=[jax.ShapeDtypeStruct((_H, _H), jnp.bfloat16)] * 4,
        compiler_params=pltpu.CompilerParams(
            dimension_semantics=("arbitrary",),
            vmem_limit_bytes=_VMEM,
        ),
    )(Xr, Xr, Xr, Xr, Xi, Xi, Xi, Xi)

    lhs1 = pl.BlockSpec((_BM, _H), lambda j, i: (i, 0))
    rhs1 = pl.BlockSpec((_H, _BN), lambda j, i: (0, j))
    out1 = pl.BlockSpec((_BM, _BN), lambda j, i: (i, j))
    pe, qo = pl.pallas_call(
        _stage1_body,
        grid=(_H // _BN, _N // _BM),
        in_specs=[lhs1, lhs1, rhs1, rhs1, rhs1, rhs1],
        out_specs=[out1, out1],
        out_shape=[jax.ShapeDtypeStruct((_N, _H), jnp.bfloat16)] * 2,
        compiler_params=pltpu.CompilerParams(
            dimension_semantics=("arbitrary", "arbitrary"),
            vmem_limit_bytes=_VMEM,
        ),
    )(_CE, _SE, F_a, F_b, G_a, G_b)

    bm2 = 1024
    lhs2 = pl.BlockSpec((bm2, _H), lambda i, j: (i, 0))
    rhs2 = pl.BlockSpec((_H, _BN), lambda i, j: (0, j))
    out2 = pl.BlockSpec((bm2, _BN), lambda i, j: (i, j))
    out = pl.pallas_call(
        _stage2_body,
        grid=(_N // bm2, _N // _BN),
        in_specs=[lhs2, lhs2, rhs2, rhs2],
        out_specs=out2,
        out_shape=jax.ShapeDtypeStruct((_N, _N), jnp.float32),
        compiler_params=pltpu.CompilerParams(
            dimension_semantics=("arbitrary", "arbitrary"),
            vmem_limit_bytes=_VMEM,
        ),
    )(pe, qo, _CTE, _STE)

    return out.reshape(1, _N, _N)
